# Initial kernel scaffold; baseline (speedup 1.0000x reference)
import jax
import jax.numpy as jnp
from jax.experimental import pallas as pl


def kernel(des, tweet, num_prop, cat_prop, new_feature, edge_index, edge_type, W_des, b_des, W_tw, b_tw, W_np, b_np, W_cp, b_cp, W_nf, b_nf, W_in, b_in, Wg1, bg1, Wg2, bg2, W_o1, b_o1, W_o2, b_o2):
    raise NotImplementedError("write your pallas kernel here")


# speedup vs baseline: 2.7463x; 2.7463x over previous
import functools

import jax
import jax.numpy as jnp
from jax.experimental import pallas as pl
from jax.experimental.pallas import tpu as pltpu

N = 10000
E = 320000
BLK = 1000
GRID = N // BLK


def _leaky(x):
    return jnp.where(x > 0, x, 0.01 * x)


def _dot(a, b):
    return jnp.dot(a, b, preferred_element_type=jnp.float32)


def _tc1_body(des_ref, tw_ref, np_ref, cp_ref, nf_ref, dp_ref,
              Wdes, bdes, Wtw, btw, Wnp, bnp, Wcp, bcp, Wnf, bnf,
              Wind, Wint, Winn, Winc, Winf, bin_, Wg1,
              g1_ref, dis_ref):
    d = _leaky(_dot(des_ref[...], Wdes[...]) + bdes[...])
    t = _leaky(_dot(tw_ref[...], Wtw[...]) + btw[...])
    n = _leaky(_dot(np_ref[...], Wnp[...]) + bnp[...])
    c = _leaky(_dot(cp_ref[...], Wcp[...]) + bcp[...])
    f = _leaky(_dot(nf_ref[...], Wnf[...]) + bnf[...])
    x = _leaky(_dot(d, Wind[...]) + _dot(t, Wint[...]) + _dot(n, Winn[...])
               + _dot(c, Winc[...]) + _dot(f, Winf[...]) + bin_[...])
    h = _dot(x, Wg1[...])
    deg = jnp.sum(dp_ref[...], axis=1) + 1.0
    dis = jax.lax.rsqrt(deg)[:, None]
    dis_ref[...] = dis
    g1_ref[...] = h * dis


def _tc1(des, tweet, num_prop, cat_prop, new_feature, deg_parts,
         W_des, b_des, W_tw, b_tw, W_np, b_np, W_cp, b_cp, W_nf, b_nf,
         W_in, b_in, Wg1, *, interpret=False):
    P = deg_parts.shape[1]
    row = lambda i: (i, 0)
    full = lambda shape: pl.BlockSpec(shape, lambda i: (0, 0))
    in_specs = [
        pl.BlockSpec((BLK, 768), row), pl.BlockSpec((BLK, 768), row),
        pl.BlockSpec((BLK, 7), row), pl.BlockSpec((BLK, 11), row),
        pl.BlockSpec((BLK, 1), row),
        pl.BlockSpec((BLK, P), row),
        full((768, 28)), full((1, 28)), full((768, 36)), full((1, 36)),
        full((7, 12)), full((1, 12)), full((11, 40)), full((1, 40)),
        full((1, 12)), full((1, 12)),
        full((28, 128)), full((36, 128)), full((12, 128)), full((40, 128)),
        full((12, 128)), full((1, 128)), full((128, 128)),
    ]
    out_specs = [pl.BlockSpec((BLK, 128), row), pl.BlockSpec((BLK, 1), row)]
    Wind = W_in[0:28]
    Wint = W_in[28:64]
    Winn = W_in[64:76]
    Winc = W_in[76:116]
    Winf = W_in[116:128]
    return pl.pallas_call(
        _tc1_body,
        grid=(GRID,),
        in_specs=in_specs,
        out_specs=out_specs,
        out_shape=[jax.ShapeDtypeStruct((N, 128), jnp.float32),
                   jax.ShapeDtypeStruct((N, 1), jnp.float32)],
        interpret=interpret,
    )(des, tweet, num_prop, cat_prop, new_feature, deg_parts,
      W_des, b_des.reshape(1, -1), W_tw, b_tw.reshape(1, -1),
      W_np, b_np.reshape(1, -1), W_cp, b_cp.reshape(1, -1),
      W_nf, b_nf.reshape(1, -1),
      Wind, Wint, Winn, Winc, Winf, b_in.reshape(1, -1), Wg1)


def _tc2_body(Sa_ref, Sb_ref, g_ref, dis_ref, bg, Wg2, g2_ref):
    dis = dis_ref[...]
    out1 = (Sa_ref[...] + Sb_ref[...] + g_ref[...]) * dis + bg[...]
    g2_ref[...] = _dot(out1, Wg2[...]) * dis


def _tc2(Sa, Sb, g1, dis, bg1, Wg2, *, interpret=False):
    row = lambda i: (i, 0)
    full = lambda shape: pl.BlockSpec(shape, lambda i: (0, 0))
    return pl.pallas_call(
        _tc2_body,
        grid=(GRID,),
        in_specs=[pl.BlockSpec((BLK, 128), row)] * 3
        + [pl.BlockSpec((BLK, 1), row), full((1, 128)), full((128, 128))],
        out_specs=pl.BlockSpec((BLK, 128), row),
        out_shape=jax.ShapeDtypeStruct((N, 128), jnp.float32),
        interpret=interpret,
    )(Sa, Sb, g1, dis, bg1.reshape(1, -1), Wg2)


def _tc3_body(Sa_ref, Sb_ref, g_ref, dis_ref, bg, Wo1, bo1, Wo2, bo2, y_ref):
    out2 = (Sa_ref[...] + Sb_ref[...] + g_ref[...]) * dis_ref[...] + bg[...]
    t = _leaky(_dot(out2, Wo1[...]) + bo1[...])
    y_ref[...] = _dot(t, Wo2[...]) + bo2[...]


def _tc3(Sa, Sb, g2, dis, bg2, W_o1, b_o1, W_o2, b_o2, *, interpret=False):
    row = lambda i: (i, 0)
    full = lambda shape: pl.BlockSpec(shape, lambda i: (0, 0))
    return pl.pallas_call(
        _tc3_body,
        grid=(GRID,),
        in_specs=[pl.BlockSpec((BLK, 128), row)] * 3
        + [pl.BlockSpec((BLK, 1), row), full((1, 128)), full((128, 128)),
           full((1, 128)), full((128, 2)), full((1, 2))],
        out_specs=pl.BlockSpec((BLK, 2), row),
        out_shape=jax.ShapeDtypeStruct((N, 2), jnp.float32),
        interpret=interpret,
    )(Sa, Sb, g2, dis, bg2.reshape(1, -1), W_o1, b_o1.reshape(1, -1),
      W_o2, b_o2.reshape(1, -1))


def kernel(des, tweet, num_prop, cat_prop, new_feature, edge_index, edge_type,
           W_des, b_des, W_tw, b_tw, W_np, b_np, W_cp, b_cp, W_nf, b_nf,
           W_in, b_in, Wg1, bg1, Wg2, bg2, W_o1, b_o1, W_o2, b_o2):
    src, dst = edge_index[0], edge_index[1]

    deg_parts = jnp.zeros((N, 1), jnp.float32).at[dst, 0].add(1.0)

    g1, dis = _tc1(des, tweet, num_prop, cat_prop, new_feature, deg_parts,
                   W_des, b_des, W_tw, b_tw, W_np, b_np, W_cp, b_cp,
                   W_nf, b_nf, W_in, b_in, Wg1)

    Z = jnp.zeros((N, 128), jnp.float32)
    S1 = jnp.zeros_like(g1).at[dst].add(g1[src])
    g2 = _tc2(S1, Z, g1, dis, bg1, Wg2)
    S2 = jnp.zeros_like(g2).at[dst].add(g2[src])
    y = _tc3(S2, Z, g2, dis, bg2, W_o1, b_o1, W_o2, b_o2)
    return y


# speedup vs baseline: 27.0085x; 9.8344x over previous
import functools

import jax
import jax.numpy as jnp
from jax import lax
from jax.experimental import pallas as pl
from jax.experimental.pallas import tpu as pltpu
from jax.experimental.pallas import tpu_sc as plsc

N = 10000
E = 320000
BLK = 1000
GRID = N // BLK

NC = 2
NS = 16
NW = NC * NS
EPT = E // NW
K = 125
CH = EPT // K
RPT = 624
ZR = 16

_sc_mesh = functools.partial(plsc.VectorSubcoreMesh,
                             core_axis_name="c", subcore_axis_name="s")


def _sc_deg_body(dst_hbm, out_hbm, acc, idx, _sem):
    wid = lax.axis_index("c") * NS + lax.axis_index("s")

    @pl.loop(0, N // 16)
    def _(i):
        acc[pl.ds(i * 16, 16)] = jnp.zeros((16,), jnp.float32)

    pltpu.sync_copy(dst_hbm.at[pl.ds(wid * EPT, EPT)], idx)

    ones = jnp.ones((16,), jnp.float32)

    @pl.loop(0, EPT // 16)
    def _(j):
        plsc.addupdate_scatter(acc, [idx[pl.ds(j * 16, 16)]], ones)

    pltpu.sync_copy(acc, out_hbm.at[wid])


def _sc_deg(dst):
    return pl.kernel(
        _sc_deg_body,
        out_type=jax.ShapeDtypeStruct((NW, N), jnp.float32),
        mesh=_sc_mesh(),
        compiler_params=pltpu.CompilerParams(needs_layout_passes=False),
        scratch_types=[
            pltpu.VMEM((N,), jnp.float32),
            pltpu.VMEM((EPT,), jnp.int32),
            pltpu.SemaphoreType.DMA,
        ],
    )(dst)


GRP = 40


def _sc_conv_body(g_hbm, srcm_hbm, dstm_hbm, out_hbm,
                  acc, sidx, didx, rows0, rows1, gsem, ssem):
    c = lax.axis_index("c")
    sid = lax.axis_index("s")
    tg = c * NS + sid
    bufs = [rows0, rows1]
    nb = len(bufs)

    @pl.loop(0, ZR)
    def _(i):
        for k in range(128 // 16):
            rows0[i, pl.ds(k * 16, 16)] = jnp.zeros((16,), jnp.float32)

    zsrc = rows0.at[pl.ds(0, ZR)]

    @pl.loop(0, RPT // ZR)
    def _(m):
        pltpu.sync_copy(zsrc, acc.at[pl.ds(sid * RPT + m * ZR, ZR)])

    @pl.when(sid == NS - 1)
    def _():
        pltpu.sync_copy(zsrc, acc.at[pl.ds(N - ZR, ZR)])

    plsc.subcore_barrier()

    @pl.loop(0, CH // GRP)
    def _(grp):
        base = tg * CH + grp * GRP
        pltpu.sync_copy(srcm_hbm.at[pl.ds(base, GRP)], sidx)
        pltpu.sync_copy(dstm_hbm.at[pl.ds(base, GRP)], didx)
        pg = [pltpu.async_copy(g_hbm.at[sidx.at[j]], bufs[j], gsem)
              for j in range(nb)]
        for j in range(GRP):
            pg[j].wait()
            pltpu.sync_copy(bufs[j % nb], acc.at[didx.at[j]], add=True)
            if j + nb < GRP:
                pg.append(pltpu.async_copy(
                    g_hbm.at[sidx.at[j + nb]], bufs[j % nb], gsem))

    plsc.subcore_barrier()
    pltpu.sync_copy(acc.at[pl.ds(sid * RPT, RPT)],
                    out_hbm.at[c, pl.ds(sid * RPT, RPT)])

    @pl.when(sid == NS - 1)
    def _():
        pltpu.sync_copy(acc.at[pl.ds(NS * RPT, N - NS * RPT)],
                        out_hbm.at[c, pl.ds(NS * RPT, N - NS * RPT)])


def _sc_conv(g, srcm, dstm):
    return pl.kernel(
        _sc_conv_body,
        out_type=jax.ShapeDtypeStruct((NC, N, 128), jnp.float32),
        mesh=_sc_mesh(),
        compiler_params=pltpu.CompilerParams(needs_layout_passes=False),
        scratch_types=[
            pltpu.VMEM_SHARED((N, 128), jnp.float32),
            pltpu.VMEM((GRP, K), jnp.int32),
            pltpu.VMEM((GRP, K), jnp.int32),
            pltpu.VMEM((K, 128), jnp.float32),
            pltpu.VMEM((K, 128), jnp.float32),
            pltpu.SemaphoreType.DMA,
            pltpu.SemaphoreType.DMA,
        ],
    )(g, srcm, dstm)


def _leaky(x):
    return jnp.where(x > 0, x, 0.01 * x)


def _dot(a, b):
    return jnp.dot(a, b, preferred_element_type=jnp.float32)


def _tc1_body(des_ref, tw_ref, np_ref, cp_ref, nf_ref, dp_ref,
              Wdes, bdes, Wtw, btw, Wnp, bnp, Wcp, bcp, Wnf, bnf,
              Wind, Wint, Winn, Winc, Winf, bin_, Wg1,
              g1_ref, dis_ref):
    bf = jnp.bfloat16
    d = _leaky(_dot(des_ref[...].astype(bf), Wdes[...]) + bdes[...])
    t = _leaky(_dot(tw_ref[...].astype(bf), Wtw[...]) + btw[...])
    n = _leaky(_dot(np_ref[...], Wnp[...]) + bnp[...])
    c = _leaky(_dot(cp_ref[...], Wcp[...]) + bcp[...])
    f = _leaky(_dot(nf_ref[...], Wnf[...]) + bnf[...])
    x = _leaky(_dot(d, Wind[...]) + _dot(t, Wint[...]) + _dot(n, Winn[...])
               + _dot(c, Winc[...]) + _dot(f, Winf[...]) + bin_[...])
    h = _dot(x, Wg1[...])
    deg = jnp.sum(dp_ref[...], axis=1) + 1.0
    dis = jax.lax.rsqrt(deg)[:, None]
    dis_ref[...] = dis
    g1_ref[...] = h * dis


def _tc1(des, tweet, num_prop, cat_prop, new_feature, deg_parts,
         W_des, b_des, W_tw, b_tw, W_np, b_np, W_cp, b_cp, W_nf, b_nf,
         W_in, b_in, Wg1, *, interpret=False):
    P = deg_parts.shape[1]
    row = lambda i: (i, 0)
    full = lambda shape: pl.BlockSpec(shape, lambda i: (0, 0))
    in_specs = [
        pl.BlockSpec((BLK, 768), row), pl.BlockSpec((BLK, 768), row),
        pl.BlockSpec((BLK, 7), row), pl.BlockSpec((BLK, 11), row),
        pl.BlockSpec((BLK, 1), row),
        pl.BlockSpec((BLK, P), row),
        full((768, 28)), full((1, 28)), full((768, 36)), full((1, 36)),
        full((7, 12)), full((1, 12)), full((11, 40)), full((1, 40)),
        full((1, 12)), full((1, 12)),
        full((28, 128)), full((36, 128)), full((12, 128)), full((40, 128)),
        full((12, 128)), full((1, 128)), full((128, 128)),
    ]
    out_specs = [pl.BlockSpec((BLK, 128), row), pl.BlockSpec((BLK, 1), row)]
    Wind = W_in[0:28]
    Wint = W_in[28:64]
    Winn = W_in[64:76]
    Winc = W_in[76:116]
    Winf = W_in[116:128]
    bf = jnp.bfloat16
    return pl.pallas_call(
        _tc1_body,
        grid=(GRID,),
        in_specs=in_specs,
        out_specs=out_specs,
        out_shape=[jax.ShapeDtypeStruct((N, 128), jnp.float32),
                   jax.ShapeDtypeStruct((N, 1), jnp.float32)],
        interpret=interpret,
    )(des, tweet, num_prop, cat_prop, new_feature,
      deg_parts,
      W_des.astype(bf), b_des.reshape(1, -1), W_tw.astype(bf),
      b_tw.reshape(1, -1),
      W_np, b_np.reshape(1, -1), W_cp, b_cp.reshape(1, -1),
      W_nf, b_nf.reshape(1, -1),
      Wind, Wint, Winn, Winc, Winf, b_in.reshape(1, -1), Wg1)


def _tc2_body(Sa_ref, Sb_ref, g_ref, dis_ref, bg, Wg2, g2_ref):
    dis = dis_ref[...]
    out1 = (Sa_ref[0] + Sb_ref[0] + g_ref[...]) * dis + bg[...]
    g2_ref[...] = _dot(out1, Wg2[...]) * dis


def _tc2(S, g1, dis, bg1, Wg2, *, interpret=False):
    row = lambda i: (i, 0)
    full = lambda shape: pl.BlockSpec(shape, lambda i: (0, 0))
    return pl.pallas_call(
        _tc2_body,
        grid=(GRID,),
        in_specs=[pl.BlockSpec((1, BLK, 128), lambda i: (0, i, 0)),
                  pl.BlockSpec((1, BLK, 128), lambda i: (1, i, 0)),
                  pl.BlockSpec((BLK, 128), row),
                  pl.BlockSpec((BLK, 1), row), full((1, 128)),
                  full((128, 128))],
        out_specs=pl.BlockSpec((BLK, 128), row),
        out_shape=jax.ShapeDtypeStruct((N, 128), jnp.float32),
        interpret=interpret,
    )(S, S, g1, dis, bg1.reshape(1, -1), Wg2)


def _tc3_body(Sa_ref, Sb_ref, g_ref, dis_ref, bg, Wo1, bo1, Wo2, bo2, y_ref):
    out2 = (Sa_ref[0] + Sb_ref[0] + g_ref[...]) * dis_ref[...] + bg[...]
    t = _leaky(_dot(out2, Wo1[...]) + bo1[...])
    y_ref[...] = _dot(t, Wo2[...]) + bo2[...]


def _tc3(S, g2, dis, bg2, W_o1, b_o1, W_o2, b_o2, *, interpret=False):
    row = lambda i: (i, 0)
    full = lambda shape: pl.BlockSpec(shape, lambda i: (0, 0))
    return pl.pallas_call(
        _tc3_body,
        grid=(GRID,),
        in_specs=[pl.BlockSpec((1, BLK, 128), lambda i: (0, i, 0)),
                  pl.BlockSpec((1, BLK, 128), lambda i: (1, i, 0)),
                  pl.BlockSpec((BLK, 128), row),
                  pl.BlockSpec((BLK, 1), row), full((1, 128)),
                  full((128, 128)), full((1, 128)), full((128, 2)),
                  full((1, 2))],
        out_specs=pl.BlockSpec((BLK, 2), row),
        out_shape=jax.ShapeDtypeStruct((N, 2), jnp.float32),
        interpret=interpret,
    )(S, S, g2, dis, bg2.reshape(1, -1), W_o1, b_o1.reshape(1, -1),
      W_o2, b_o2.reshape(1, -1))


def kernel(des, tweet, num_prop, cat_prop, new_feature, edge_index, edge_type,
           W_des, b_des, W_tw, b_tw, W_np, b_np, W_cp, b_cp, W_nf, b_nf,
           W_in, b_in, Wg1, bg1, Wg2, bg2, W_o1, b_o1, W_o2, b_o2):
    src, dst = edge_index[0], edge_index[1]
    srcm = src.reshape(NW * CH, K)
    dstm = dst.reshape(NW * CH, K)

    deg_parts = _sc_deg(dst).T

    g1, dis = _tc1(des, tweet, num_prop, cat_prop, new_feature, deg_parts,
                   W_des, b_des, W_tw, b_tw, W_np, b_np, W_cp, b_cp,
                   W_nf, b_nf, W_in, b_in, Wg1)

    S1 = _sc_conv(g1, srcm, dstm)
    g2 = _tc2(S1, g1, dis, bg1, Wg2)
    S2 = _sc_conv(g2, srcm, dstm)
    y = _tc3(S2, g2, dis, bg2, W_o1, b_o1, W_o2, b_o2)
    return y


# speedup vs baseline: 27.5324x; 1.0194x over previous
import functools

import jax
import jax.numpy as jnp
from jax import lax
from jax.experimental import pallas as pl
from jax.experimental.pallas import tpu as pltpu
from jax.experimental.pallas import tpu_sc as plsc

N = 10000
E = 320000
BLK = 2000
GRID = N // BLK

NC = 2
NS = 16
NW = NC * NS
EPT = E // NW
K = 125
CH = EPT // K
RPT = 624
ZR = 16

_sc_mesh = functools.partial(plsc.VectorSubcoreMesh,
                             core_axis_name="c", subcore_axis_name="s")


def _sc_deg_body(dst_hbm, out_hbm, acc, idx, _sem):
    wid = lax.axis_index("c") * NS + lax.axis_index("s")

    @pl.loop(0, N // 16)
    def _(i):
        acc[pl.ds(i * 16, 16)] = jnp.zeros((16,), jnp.float32)

    pltpu.sync_copy(dst_hbm.at[pl.ds(wid * EPT, EPT)], idx)

    ones = jnp.ones((16,), jnp.float32)

    @pl.loop(0, EPT // 16)
    def _(j):
        plsc.addupdate_scatter(acc, [idx[pl.ds(j * 16, 16)]], ones)

    pltpu.sync_copy(acc, out_hbm.at[wid])


def _sc_deg(dst):
    return pl.kernel(
        _sc_deg_body,
        out_type=jax.ShapeDtypeStruct((NW, N), jnp.float32),
        mesh=_sc_mesh(),
        compiler_params=pltpu.CompilerParams(needs_layout_passes=False),
        scratch_types=[
            pltpu.VMEM((N,), jnp.float32),
            pltpu.VMEM((EPT,), jnp.int32),
            pltpu.SemaphoreType.DMA,
        ],
    )(dst)


GRP = 40


def _sc_conv_body(g_hbm, srcm_hbm, dstm_hbm, out_hbm,
                  acc, sidx, didx, rows0, rows1, gsem, ssem):
    c = lax.axis_index("c")
    sid = lax.axis_index("s")
    tg = c * NS + sid
    bufs = [rows0, rows1]
    nb = len(bufs)

    @pl.loop(0, ZR)
    def _(i):
        for k in range(128 // 16):
            rows0[i, pl.ds(k * 16, 16)] = jnp.zeros((16,), jnp.float32)

    zsrc = rows0.at[pl.ds(0, ZR)]

    @pl.loop(0, RPT // ZR)
    def _(m):
        pltpu.sync_copy(zsrc, acc.at[pl.ds(sid * RPT + m * ZR, ZR)])

    @pl.when(sid == NS - 1)
    def _():
        pltpu.sync_copy(zsrc, acc.at[pl.ds(N - ZR, ZR)])

    plsc.subcore_barrier()

    @pl.loop(0, CH // GRP)
    def _(grp):
        base = tg * CH + grp * GRP
        pltpu.sync_copy(srcm_hbm.at[pl.ds(base, GRP)], sidx)
        pltpu.sync_copy(dstm_hbm.at[pl.ds(base, GRP)], didx)
        pg = [pltpu.async_copy(g_hbm.at[sidx.at[j]], bufs[j], gsem)
              for j in range(nb)]
        for j in range(GRP):
            pg[j].wait()
            pltpu.sync_copy(bufs[j % nb], acc.at[didx.at[j]], add=True)
            if j + nb < GRP:
                pg.append(pltpu.async_copy(
                    g_hbm.at[sidx.at[j + nb]], bufs[j % nb], gsem))

    plsc.subcore_barrier()
    pltpu.sync_copy(acc.at[pl.ds(sid * RPT, RPT)],
                    out_hbm.at[c, pl.ds(sid * RPT, RPT)])

    @pl.when(sid == NS - 1)
    def _():
        pltpu.sync_copy(acc.at[pl.ds(NS * RPT, N - NS * RPT)],
                        out_hbm.at[c, pl.ds(NS * RPT, N - NS * RPT)])


def _sc_conv(g, srcm, dstm):
    return pl.kernel(
        _sc_conv_body,
        out_type=jax.ShapeDtypeStruct((NC, N, 128), jnp.float32),
        mesh=_sc_mesh(),
        compiler_params=pltpu.CompilerParams(needs_layout_passes=False),
        scratch_types=[
            pltpu.VMEM_SHARED((N, 128), jnp.float32),
            pltpu.VMEM((GRP, K), jnp.int32),
            pltpu.VMEM((GRP, K), jnp.int32),
            pltpu.VMEM((K, 128), jnp.float32),
            pltpu.VMEM((K, 128), jnp.float32),
            pltpu.SemaphoreType.DMA,
            pltpu.SemaphoreType.DMA,
        ],
    )(g, srcm, dstm)


def _leaky(x):
    return jnp.where(x > 0, x, 0.01 * x)


def _dot(a, b):
    return jnp.dot(a, b, preferred_element_type=jnp.float32)


def _tc1_body(des_ref, tw_ref, np_ref, cp_ref, nf_ref, dp_ref,
              Wdes, bdes, Wtw, btw, Wnp, bnp, Wcp, bcp, Wnf, bnf,
              Wind, Wint, Winn, Winc, Winf, bin_, Wg1,
              g1_ref, dis_ref):
    bf = jnp.bfloat16
    d = _leaky(_dot(des_ref[...].astype(bf), Wdes[...]) + bdes[...])
    t = _leaky(_dot(tw_ref[...].astype(bf), Wtw[...]) + btw[...])
    n = _leaky(_dot(np_ref[...], Wnp[...]) + bnp[...])
    c = _leaky(_dot(cp_ref[...], Wcp[...]) + bcp[...])
    f = _leaky(_dot(nf_ref[...], Wnf[...]) + bnf[...])
    x = _leaky(_dot(d, Wind[...]) + _dot(t, Wint[...]) + _dot(n, Winn[...])
               + _dot(c, Winc[...]) + _dot(f, Winf[...]) + bin_[...])
    h = _dot(x, Wg1[...])
    deg = jnp.sum(dp_ref[...], axis=1) + 1.0
    dis = jax.lax.rsqrt(deg)[:, None]
    dis_ref[...] = dis
    g1_ref[...] = h * dis


def _tc1(des, tweet, num_prop, cat_prop, new_feature, deg_parts,
         W_des, b_des, W_tw, b_tw, W_np, b_np, W_cp, b_cp, W_nf, b_nf,
         W_in, b_in, Wg1, *, interpret=False):
    P = deg_parts.shape[1]
    row = lambda i: (i, 0)
    full = lambda shape: pl.BlockSpec(shape, lambda i: (0, 0))
    in_specs = [
        pl.BlockSpec((BLK, 768), row), pl.BlockSpec((BLK, 768), row),
        pl.BlockSpec((BLK, 7), row), pl.BlockSpec((BLK, 11), row),
        pl.BlockSpec((BLK, 1), row),
        pl.BlockSpec((BLK, P), row),
        full((768, 28)), full((1, 28)), full((768, 36)), full((1, 36)),
        full((7, 12)), full((1, 12)), full((11, 40)), full((1, 40)),
        full((1, 12)), full((1, 12)),
        full((28, 128)), full((36, 128)), full((12, 128)), full((40, 128)),
        full((12, 128)), full((1, 128)), full((128, 128)),
    ]
    out_specs = [pl.BlockSpec((BLK, 128), row), pl.BlockSpec((BLK, 1), row)]
    Wind = W_in[0:28]
    Wint = W_in[28:64]
    Winn = W_in[64:76]
    Winc = W_in[76:116]
    Winf = W_in[116:128]
    bf = jnp.bfloat16
    return pl.pallas_call(
        _tc1_body,
        grid=(GRID,),
        in_specs=in_specs,
        out_specs=out_specs,
        out_shape=[jax.ShapeDtypeStruct((N, 128), jnp.float32),
                   jax.ShapeDtypeStruct((N, 1), jnp.float32)],
        interpret=interpret,
    )(des, tweet, num_prop, cat_prop, new_feature,
      deg_parts,
      W_des.astype(bf), b_des.reshape(1, -1), W_tw.astype(bf),
      b_tw.reshape(1, -1),
      W_np, b_np.reshape(1, -1), W_cp, b_cp.reshape(1, -1),
      W_nf, b_nf.reshape(1, -1),
      Wind, Wint, Winn, Winc, Winf, b_in.reshape(1, -1), Wg1)


def _tc2_body(Sa_ref, Sb_ref, g_ref, dis_ref, bg, Wg2, g2_ref):
    dis = dis_ref[...]
    out1 = (Sa_ref[0] + Sb_ref[0] + g_ref[...]) * dis + bg[...]
    g2_ref[...] = _dot(out1, Wg2[...]) * dis


def _tc2(S, g1, dis, bg1, Wg2, *, interpret=False):
    row = lambda i: (i, 0)
    full = lambda shape: pl.BlockSpec(shape, lambda i: (0, 0))
    return pl.pallas_call(
        _tc2_body,
        grid=(GRID,),
        in_specs=[pl.BlockSpec((1, BLK, 128), lambda i: (0, i, 0)),
                  pl.BlockSpec((1, BLK, 128), lambda i: (1, i, 0)),
                  pl.BlockSpec((BLK, 128), row),
                  pl.BlockSpec((BLK, 1), row), full((1, 128)),
                  full((128, 128))],
        out_specs=pl.BlockSpec((BLK, 128), row),
        out_shape=jax.ShapeDtypeStruct((N, 128), jnp.float32),
        interpret=interpret,
    )(S, S, g1, dis, bg1.reshape(1, -1), Wg2)


def _tc3_body(Sa_ref, Sb_ref, g_ref, dis_ref, bg, Wo1, bo1, Wo2, bo2, y_ref):
    out2 = (Sa_ref[0] + Sb_ref[0] + g_ref[...]) * dis_ref[...] + bg[...]
    t = _leaky(_dot(out2, Wo1[...]) + bo1[...])
    y_ref[...] = _dot(t, Wo2[...]) + bo2[...]


def _tc3(S, g2, dis, bg2, W_o1, b_o1, W_o2, b_o2, *, interpret=False):
    row = lambda i: (i, 0)
    full = lambda shape: pl.BlockSpec(shape, lambda i: (0, 0))
    return pl.pallas_call(
        _tc3_body,
        grid=(GRID,),
        in_specs=[pl.BlockSpec((1, BLK, 128), lambda i: (0, i, 0)),
                  pl.BlockSpec((1, BLK, 128), lambda i: (1, i, 0)),
                  pl.BlockSpec((BLK, 128), row),
                  pl.BlockSpec((BLK, 1), row), full((1, 128)),
                  full((128, 128)), full((1, 128)), full((128, 2)),
                  full((1, 2))],
        out_specs=pl.BlockSpec((BLK, 2), row),
        out_shape=jax.ShapeDtypeStruct((N, 2), jnp.float32),
        interpret=interpret,
    )(S, S, g2, dis, bg2.reshape(1, -1), W_o1, b_o1.reshape(1, -1),
      W_o2, b_o2.reshape(1, -1))


def kernel(des, tweet, num_prop, cat_prop, new_feature, edge_index, edge_type,
           W_des, b_des, W_tw, b_tw, W_np, b_np, W_cp, b_cp, W_nf, b_nf,
           W_in, b_in, Wg1, bg1, Wg2, bg2, W_o1, b_o1, W_o2, b_o2):
    src, dst = edge_index[0], edge_index[1]
    srcm = src.reshape(NW * CH, K)
    dstm = dst.reshape(NW * CH, K)

    deg_parts = _sc_deg(dst).T

    g1, dis = _tc1(des, tweet, num_prop, cat_prop, new_feature, deg_parts,
                   W_des, b_des, W_tw, b_tw, W_np, b_np, W_cp, b_cp,
                   W_nf, b_nf, W_in, b_in, Wg1)

    S1 = _sc_conv(g1, srcm, dstm)
    g2 = _tc2(S1, g1, dis, bg1, Wg2)
    S2 = _sc_conv(g2, srcm, dstm)
    y = _tc3(S2, g2, dis, bg2, W_o1, b_o1, W_o2, b_o2)
    return y
